# baseline (speedup 1.0000x reference)
import functools

import jax
import jax.numpy as jnp
from jax import lax
from jax.experimental import pallas as pl
from jax.experimental.pallas import tpu as pltpu
from jax.experimental.pallas import tpu_sc as plsc

_NB = 8
_CP = 64


def _proj_kernel(B, T, E, H, F,
                 vf_ref, twf_ref, Wv_ref, Wtw_ref, Wcw_ref,
                 wi_ref, ptw_ref):
    vf = vf_ref[...]
    vh = lax.dot_general(vf, Wv_ref[...],
                         (((1,), (1,)), ((), ())),
                         preferred_element_type=jnp.float32)
    twf = twf_ref[...].reshape(B * T, E)
    twh = lax.dot_general(twf, Wtw_ref[...],
                          (((1,), (1,)), ((), ())),
                          preferred_element_type=jnp.float32)
    wi_logits = jnp.sum(twh.reshape(B, T, H) * vh[:, None, :], axis=-1)
    m = jnp.max(wi_logits, axis=1, keepdims=True)
    e = jnp.exp(wi_logits - m)
    wi = e / jnp.sum(e, axis=1, keepdims=True)
    wi_ref[...] = wi[:, :, None]
    ptw = lax.dot_general(twh, Wcw_ref[...],
                          (((1,), (0,)), ((), ())),
                          preferred_element_type=jnp.float32)
    ptw_ref[...] = ptw.reshape(B, T, E)


def _sc_si_kernel(B, T, C, E, PER_TILE,
                  cand_hbm, ptw_hbm, si_hbm,
                  ptw_v, buf0, buf1, si_smem, tmp_v, si_v, sem0, sem1):
    CP = _CP
    nc = jax.lax.axis_index("c")
    ns = jax.lax.axis_index("s")
    wid = ns * 2 + nc
    EC = E // 16
    lanes = lax.iota(jnp.int32, 16)

    for bi in range(PER_TILE):
        b = wid * PER_TILE + bi
        pltpu.sync_copy(ptw_hbm.at[b], ptw_v)
        pltpu.async_copy(cand_hbm.at[b, 0], buf0, sem0)
        pltpu.async_copy(cand_hbm.at[b, 1], buf1, sem1)

        def t2body(t2, carry):
            for par in range(2):
                t = t2 * 2 + par
                buf = (buf0, buf1)[par]
                sem = (sem0, sem1)[par]
                pltpu.make_async_copy(cand_hbm.at[b, t], buf, sem).wait()

                def cbody(c, carry2):
                    ptwk = [ptw_v[t, pl.ds(k * 16, 16)] for k in range(EC)]
                    acc = buf[c, pl.ds(0, 16)] * ptwk[0]
                    for k in range(1, EC):
                        acc = acc + buf[c, pl.ds(k * 16, 16)] * ptwk[k]
                    s = acc[0]
                    for l in range(1, 16):
                        s = s + acc[l]
                    si_smem[c] = s
                    return carry2

                lax.fori_loop(0, C, cbody, 0, unroll=2)
                for j in range(CP // 16):
                    v = jnp.zeros((16,), jnp.float32)
                    for l in range(16):
                        c = j * 16 + l
                        if c < C:
                            v = jnp.where(lanes == l, si_smem[c], v)
                    si_v[t, pl.ds(j * 16, 16)] = v

                @pl.when(t < T - 2)
                def _prefetch():
                    pltpu.async_copy(cand_hbm.at[b, t + 2], buf, sem)
            return carry

        lax.fori_loop(0, T // 2, t2body, 0)
        pltpu.sync_copy(si_v, si_hbm.at[b])


def _combine_kernel(B, T, C, L, D,
                    si_ref, wi_ref, ctx_ref, pmask_ref, twp_ref, cwp_ref,
                    asf_ref, pctx_ref, sti_ref):
    NB = _NB
    si = si_ref[...][:, :, :C]
    m = jnp.max(si, axis=2, keepdims=True)
    e = jnp.exp(si - m)
    sub = e / jnp.sum(e, axis=2, keepdims=True)
    score = wi_ref[...] * sub
    masked = jnp.where(pmask_ref[...] != 0, -jnp.inf, score)

    it = lax.broadcasted_iota(jnp.int32, (NB, T, C), 1)
    ic = lax.broadcasted_iota(jnp.int32, (NB, T, C), 2)
    flat_idx = it * C + ic
    gmax = jnp.max(masked, axis=(1, 2), keepdims=True)
    am = jnp.min(jnp.where(masked == gmax, flat_idx, T * C),
                 axis=(1, 2), keepdims=True)
    ti = am // C
    ci = am - ti * C

    iota_t = lax.broadcasted_iota(jnp.int32, (NB, 1, T), 2)
    tpos = jnp.sum(jnp.where(iota_t == ti, twp_ref[...], 0),
                   axis=(1, 2), keepdims=True)
    cpos = jnp.sum(jnp.where((it == ti) & (ic == ci), cwp_ref[...], 0),
                   axis=(1, 2), keepdims=True)
    valid = (tpos < L - 1) & (cpos < L - 1)

    ctx_g = ctx_ref[...]
    riota = lax.broadcasted_iota(jnp.int32, (NB, L, 1), 1)
    src_row = jnp.sum(jnp.where(riota == cpos, ctx_g, 0.0),
                      axis=1, keepdims=True)
    wmask = (riota == tpos) & valid
    pctx_ref[...] = jnp.where(wmask, src_row, ctx_g)
    asf_ref[...] = masked
    sti_ref[...] = jnp.broadcast_to(ti, (NB, 1, 8))


def kernel(visual_feature, target_word_feature, candidate_word_feature, ctx,
           perturb_mask, target_word_position, candidate_word_position,
           Wv, Wtw, Wcw):
    B, F = visual_feature.shape
    _, T, E = target_word_feature.shape
    C = candidate_word_feature.shape[2]
    _, L, D = ctx.shape
    H = Wv.shape[0]
    NB = _NB
    CP = _CP

    cwf3 = candidate_word_feature.reshape(B, T * C, E)
    pmask3 = perturb_mask.reshape(B, T, C).astype(jnp.int32)
    twp3 = target_word_position.astype(jnp.int32).reshape(B, 1, T)
    cwp = candidate_word_position.astype(jnp.int32)

    wi, ptw = pl.pallas_call(
        functools.partial(_proj_kernel, B, T, E, H, F),
        in_specs=[
            pl.BlockSpec((B, F), lambda: (0, 0)),
            pl.BlockSpec((B, T, E), lambda: (0, 0, 0)),
            pl.BlockSpec((H, F), lambda: (0, 0)),
            pl.BlockSpec((H, E), lambda: (0, 0)),
            pl.BlockSpec((H, E), lambda: (0, 0)),
        ],
        out_specs=[
            pl.BlockSpec((B, T, 1), lambda: (0, 0, 0)),
            pl.BlockSpec((B, T, E), lambda: (0, 0, 0)),
        ],
        out_shape=[
            jax.ShapeDtypeStruct((B, T, 1), jnp.float32),
            jax.ShapeDtypeStruct((B, T, E), jnp.float32),
        ],
    )(visual_feature, target_word_feature, Wv, Wtw, Wcw)

    info = plsc.get_sparse_core_info()
    num_tiles = info.num_cores * info.num_subcores
    per_tile = B // num_tiles
    mesh = plsc.VectorSubcoreMesh(core_axis_name="c", subcore_axis_name="s")
    si64 = pl.kernel(
        functools.partial(_sc_si_kernel, B, T, C, E, per_tile),
        mesh=mesh,
        out_type=jax.ShapeDtypeStruct((B, T, CP), jnp.float32),
        scratch_types=[
            pltpu.VMEM((T, E), jnp.float32),
            pltpu.VMEM((C, E), jnp.float32),
            pltpu.VMEM((C, E), jnp.float32),
            pltpu.SMEM((C,), jnp.float32),
            pltpu.VMEM((16,), jnp.float32),
            pltpu.VMEM((T, CP), jnp.float32),
            pltpu.SemaphoreType.DMA,
            pltpu.SemaphoreType.DMA,
        ],
    )(candidate_word_feature, ptw)

    asf3, pctx, sti3 = pl.pallas_call(
        functools.partial(_combine_kernel, B, T, C, L, D),
        grid=(B // NB,),
        in_specs=[
            pl.BlockSpec((NB, T, CP), lambda g: (g, 0, 0)),
            pl.BlockSpec((NB, T, 1), lambda g: (g, 0, 0)),
            pl.BlockSpec((NB, L, D), lambda g: (g, 0, 0)),
            pl.BlockSpec((NB, T, C), lambda g: (g, 0, 0)),
            pl.BlockSpec((NB, 1, T), lambda g: (g, 0, 0)),
            pl.BlockSpec((NB, T, C), lambda g: (g, 0, 0)),
        ],
        out_specs=[
            pl.BlockSpec((NB, T, C), lambda g: (g, 0, 0)),
            pl.BlockSpec((NB, L, D), lambda g: (g, 0, 0)),
            pl.BlockSpec((NB, 1, 8), lambda g: (g, 0, 0)),
        ],
        out_shape=[
            jax.ShapeDtypeStruct((B, T, C), jnp.float32),
            jax.ShapeDtypeStruct((B, L, D), jnp.float32),
            jax.ShapeDtypeStruct((B, 1, 8), jnp.int32),
        ],
    )(si64, wi, ctx, pmask3, twp3, cwp)

    return (asf3.reshape(B, T * C), pctx, sti3[:, 0, 0])
